# Initial kernel scaffold; baseline (speedup 1.0000x reference)
import jax
import jax.numpy as jnp
from jax.experimental import pallas as pl


def kernel(x, edge_index, edge_types, relation_weights, self_weight, bias):
    raise NotImplementedError("write your pallas kernel here")


# speedup vs baseline: 38.0781x; 38.0781x over previous
import functools

import jax
import jax.numpy as jnp
from jax import lax
from jax.experimental import pallas as pl
from jax.experimental.pallas import tpu as pltpu
from jax.experimental.pallas import tpu_sc as plsc

N_NODES = 10000
N_EDGES = 320000
D = 128
R = 8

BN = 1000
EB = 250

CHUNK = 128
NUM_CHUNKS = N_EDGES // CHUNK
NUM_WORKERS = 32
BASE_CHUNKS = NUM_CHUNKS // NUM_WORKERS
EXTRA_CHUNKS = NUM_CHUNKS - BASE_CHUNKS * NUM_WORKERS
N_PAD = 10112
ROWS_PER_TILE = N_PAD // 16


def _tc_prep_body(x_ref, w_ref, bias_ref, h_ref, self_ref):
    xb = x_ref[...]
    for r in range(R):
        h_ref[r] = jnp.dot(xb, w_ref[r], preferred_element_type=jnp.float32)
    self_ref[...] = (
        jnp.dot(xb, w_ref[R], preferred_element_type=jnp.float32) + bias_ref[...]
    )


_tc_prep = pl.pallas_call(
    _tc_prep_body,
    grid=(N_NODES // BN,),
    in_specs=[
        pl.BlockSpec((BN, D), lambda i: (i, 0)),
        pl.BlockSpec((R + 1, D, D), lambda i: (0, 0, 0)),
        pl.BlockSpec((1, D), lambda i: (0, 0)),
    ],
    out_specs=[
        pl.BlockSpec((R, BN, D), lambda i: (0, i, 0)),
        pl.BlockSpec((BN, D), lambda i: (i, 0)),
    ],
    out_shape=[
        jax.ShapeDtypeStruct((R, N_NODES, D), jnp.float32),
        jax.ShapeDtypeStruct((N_NODES, D), jnp.float32),
    ],
)


def _tc_gidx_body(t_ref, s_ref, g_ref):
    g_ref[...] = t_ref[...] * N_NODES + s_ref[...]


_tc_gidx = pl.pallas_call(
    _tc_gidx_body,
    out_shape=jax.ShapeDtypeStruct((N_EDGES // 128, 128), jnp.int32),
)


def _tc_combine_body(acc_ref, self_ref, o_ref):
    o_ref[...] = jnp.maximum(self_ref[...] + acc_ref[0] + acc_ref[1], 0.0)


_tc_combine = pl.pallas_call(
    _tc_combine_body,
    grid=(N_NODES // BN,),
    in_specs=[
        pl.BlockSpec((2, BN, D), lambda i: (0, i, 0)),
        pl.BlockSpec((BN, D), lambda i: (i, 0)),
    ],
    out_specs=pl.BlockSpec((BN, D), lambda i: (i, 0)),
    out_shape=jax.ShapeDtypeStruct((N_NODES, D), jnp.float32),
)


@functools.cache
def _make_sc_scatter():
    @functools.partial(
        pl.kernel,
        mesh=plsc.VectorSubcoreMesh(core_axis_name="c", subcore_axis_name="s"),
        out_type=jax.ShapeDtypeStruct((2, N_PAD, D), jnp.float32),
        scratch_types=[
            pltpu.VMEM((CHUNK,), jnp.int32),
            pltpu.VMEM((CHUNK,), jnp.int32),
            pltpu.VMEM((CHUNK, D), jnp.float32),
            pltpu.VMEM_SHARED((N_PAD, D), jnp.float32),
            pltpu.SemaphoreType.DMA,
        ],
    )
    def _sc_scatter(h_hbm, g_hbm, dst_hbm, zeros_hbm, acc_hbm,
                    idx_v, dst_v, rows_v, acc_sh, sem):
        c = lax.axis_index("c")
        s = lax.axis_index("s")
        wid = s * 2 + c

        pltpu.sync_copy(
            zeros_hbm.at[pl.ds(s * ROWS_PER_TILE, ROWS_PER_TILE)],
            acc_sh.at[pl.ds(s * ROWS_PER_TILE, ROWS_PER_TILE)],
        )
        plsc.subcore_barrier()

        start = wid * BASE_CHUNKS + jnp.minimum(wid, EXTRA_CHUNKS)
        count = BASE_CHUNKS + jnp.where(wid < EXTRA_CHUNKS, 1, 0)

        def body(j, carry):
            e0 = (start + j) * CHUNK
            pltpu.sync_copy(g_hbm.at[pl.ds(e0, CHUNK)], idx_v)
            pltpu.sync_copy(dst_hbm.at[pl.ds(e0, CHUNK)], dst_v)
            pltpu.async_copy(h_hbm.at[idx_v], rows_v, sem).wait()
            pltpu.sync_copy(rows_v, acc_sh.at[dst_v], add=True)
            return carry

        lax.fori_loop(0, count, body, 0)

        plsc.subcore_barrier()
        pltpu.sync_copy(
            acc_sh.at[pl.ds(s * ROWS_PER_TILE, ROWS_PER_TILE)],
            acc_hbm.at[c].at[pl.ds(s * ROWS_PER_TILE, ROWS_PER_TILE)],
        )

    return _sc_scatter


def kernel(x, edge_index, edge_types, relation_weights, self_weight, bias):
    src = edge_index[0]
    dst = edge_index[1]
    w_all = jnp.concatenate([relation_weights, self_weight[None]], axis=0)
    bias2d = bias.reshape(1, D)
    t2d = edge_types.reshape(-1, 128)
    s2d = src.reshape(-1, 128)

    h, self_msg = _tc_prep(x, w_all, bias2d)
    g2d = _tc_gidx(t2d, s2d)
    h_flat = h.reshape(R * N_NODES, D)
    g = g2d.reshape(-1)
    zeros = jnp.zeros((N_PAD, D), jnp.float32)

    acc = _make_sc_scatter()(h_flat, g, dst, zeros)
    return _tc_combine(acc, self_msg)


# speedup vs baseline: 63.5691x; 1.6694x over previous
import functools

import jax
import jax.numpy as jnp
from jax import lax
from jax.experimental import pallas as pl
from jax.experimental.pallas import tpu as pltpu
from jax.experimental.pallas import tpu_sc as plsc

N_NODES = 10000
N_EDGES = 320000
D = 128
R = 8

BN = 1000
EB = 250

CHUNK = 128
NUM_CHUNKS = N_EDGES // CHUNK
NUM_WORKERS = 32
BASE_CHUNKS = NUM_CHUNKS // NUM_WORKERS
EXTRA_CHUNKS = NUM_CHUNKS - BASE_CHUNKS * NUM_WORKERS
N_PAD = 10112
ROWS_PER_TILE = N_PAD // 16


def _tc_prep_body(x_ref, w_ref, bias_ref, h_ref, self_ref):
    xb = x_ref[...]
    for r in range(R):
        h_ref[r] = jnp.dot(xb, w_ref[r], preferred_element_type=jnp.float32)
    self_ref[...] = (
        jnp.dot(xb, w_ref[R], preferred_element_type=jnp.float32) + bias_ref[...]
    )


_tc_prep = pl.pallas_call(
    _tc_prep_body,
    grid=(N_NODES // BN,),
    in_specs=[
        pl.BlockSpec((BN, D), lambda i: (i, 0)),
        pl.BlockSpec((R + 1, D, D), lambda i: (0, 0, 0)),
        pl.BlockSpec((1, D), lambda i: (0, 0)),
    ],
    out_specs=[
        pl.BlockSpec((R, BN, D), lambda i: (0, i, 0)),
        pl.BlockSpec((BN, D), lambda i: (i, 0)),
    ],
    out_shape=[
        jax.ShapeDtypeStruct((R, N_NODES, D), jnp.float32),
        jax.ShapeDtypeStruct((N_NODES, D), jnp.float32),
    ],
)


def _tc_gidx_body(t_ref, s_ref, g_ref):
    g_ref[...] = t_ref[...] * N_NODES + s_ref[...]


_tc_gidx = pl.pallas_call(
    _tc_gidx_body,
    out_shape=jax.ShapeDtypeStruct((N_EDGES // 128, 128), jnp.int32),
)


def _tc_combine_body(acc_ref, self_ref, o_ref):
    o_ref[...] = jnp.maximum(self_ref[...] + acc_ref[0] + acc_ref[1], 0.0)


_tc_combine = pl.pallas_call(
    _tc_combine_body,
    grid=(N_NODES // BN,),
    in_specs=[
        pl.BlockSpec((2, BN, D), lambda i: (0, i, 0)),
        pl.BlockSpec((BN, D), lambda i: (i, 0)),
    ],
    out_specs=pl.BlockSpec((BN, D), lambda i: (i, 0)),
    out_shape=jax.ShapeDtypeStruct((N_NODES, D), jnp.float32),
)


@functools.cache
def _make_sc_scatter():
    @functools.partial(
        pl.kernel,
        mesh=plsc.VectorSubcoreMesh(core_axis_name="c", subcore_axis_name="s"),
        out_type=jax.ShapeDtypeStruct((2, N_PAD, D), jnp.float32),
        scratch_types=[
            pltpu.VMEM((CHUNK,), jnp.int32),
            pltpu.VMEM((CHUNK,), jnp.int32),
            pltpu.VMEM((CHUNK,), jnp.int32),
            pltpu.VMEM((CHUNK,), jnp.int32),
            pltpu.VMEM((CHUNK,), jnp.int32),
            pltpu.VMEM((CHUNK,), jnp.int32),
            pltpu.VMEM((CHUNK, D), jnp.float32),
            pltpu.VMEM((CHUNK, D), jnp.float32),
            pltpu.VMEM((CHUNK, D), jnp.float32),
            pltpu.VMEM_SHARED((N_PAD, D), jnp.float32),
            pltpu.SemaphoreType.DMA,
            pltpu.SemaphoreType.DMA,
            pltpu.SemaphoreType.DMA,
            pltpu.SemaphoreType.DMA,
            pltpu.SemaphoreType.DMA,
            pltpu.SemaphoreType.DMA,
        ],
    )
    def _sc_scatter(h_hbm, g_hbm, dst_hbm, zeros_hbm, acc_hbm,
                    g0, g1, g2, d0, d1, d2, r0, r1, r2, acc_sh,
                    gs0, gs1, gs2, ss0, ss1, ss2):
        G = (g0, g1, g2)
        Dx = (d0, d1, d2)
        Rw = (r0, r1, r2)
        GS = (gs0, gs1, gs2)
        SS = (ss0, ss1, ss2)

        c = lax.axis_index("c")
        s = lax.axis_index("s")
        wid = s * 2 + c

        pltpu.sync_copy(
            zeros_hbm.at[pl.ds(s * ROWS_PER_TILE, ROWS_PER_TILE)],
            acc_sh.at[pl.ds(s * ROWS_PER_TILE, ROWS_PER_TILE)],
        )
        plsc.subcore_barrier()

        start = wid * BASE_CHUNKS
        base = start * CHUNK

        def load_g(j, sl):
            pltpu.sync_copy(g_hbm.at[pl.ds((start + j) * CHUNK, CHUNK)], G[sl])

        def load_d(j, sl):
            pltpu.sync_copy(dst_hbm.at[pl.ds((start + j) * CHUNK, CHUNK)], Dx[sl])

        def fire_gather(sl):
            pltpu.async_copy(h_hbm.at[G[sl]], Rw[sl], GS[sl])

        def wait_gather(sl):
            pltpu.make_async_copy(h_hbm.at[G[sl]], Rw[sl], GS[sl]).wait()

        def fire_scatter(sl):
            pltpu.async_copy(Rw[sl], acc_sh.at[Dx[sl]], SS[sl], add=True)

        def wait_scatter(sl):
            pltpu.make_async_copy(Rw[sl], acc_sh.at[Dx[sl]], SS[sl]).wait()

        load_g(0, 0)
        load_g(1, 1)
        load_g(2, 2)
        load_d(0, 0)
        load_d(1, 1)
        fire_gather(0)
        fire_gather(1)
        wait_gather(0)
        fire_scatter(0)
        load_d(2, 2)
        fire_gather(2)
        wait_gather(1)
        fire_scatter(1)
        load_g(3, 0)

        def steady(j, sl):
            sp1 = (sl + 1) % 3
            sp2 = (sl + 2) % 3
            wait_scatter(sp1)
            load_d(j + 1, sp1)
            fire_gather(sp1)
            wait_gather(sl)
            fire_scatter(sl)
            load_g(j + 2, sp2)

        def loop_body(jj, carry):
            j = 3 * jj + 2
            steady(j, 2)
            steady(j + 1, 0)
            steady(j + 2, 1)
            return carry

        lax.fori_loop(0, (BASE_CHUNKS - 3) // 3, loop_body, 0)

        wait_scatter(0)
        wait_gather(2)
        fire_scatter(2)
        wait_scatter(1)
        wait_scatter(2)

        @pl.when(wid < EXTRA_CHUNKS)
        def _():
            e0 = (NUM_CHUNKS - EXTRA_CHUNKS + wid) * CHUNK
            pltpu.sync_copy(g_hbm.at[pl.ds(e0, CHUNK)], G[0])
            pltpu.sync_copy(dst_hbm.at[pl.ds(e0, CHUNK)], Dx[0])
            pltpu.async_copy(h_hbm.at[G[0]], Rw[0], GS[0]).wait()
            pltpu.sync_copy(Rw[0], acc_sh.at[Dx[0]], add=True)

        plsc.subcore_barrier()
        pltpu.sync_copy(
            acc_sh.at[pl.ds(s * ROWS_PER_TILE, ROWS_PER_TILE)],
            acc_hbm.at[c].at[pl.ds(s * ROWS_PER_TILE, ROWS_PER_TILE)],
        )

    return _sc_scatter


def kernel(x, edge_index, edge_types, relation_weights, self_weight, bias):
    src = edge_index[0]
    dst = edge_index[1]
    w_all = jnp.concatenate([relation_weights, self_weight[None]], axis=0)
    bias2d = bias.reshape(1, D)
    t2d = edge_types.reshape(-1, 128)
    s2d = src.reshape(-1, 128)

    h, self_msg = _tc_prep(x, w_all, bias2d)
    g2d = _tc_gidx(t2d, s2d)
    h_flat = h.reshape(R * N_NODES, D)
    g = g2d.reshape(-1)
    zeros = jnp.zeros((N_PAD, D), jnp.float32)

    acc = _make_sc_scatter()(h_flat, g, dst, zeros)
    return _tc_combine(acc, self_msg)


# speedup vs baseline: 68.4659x; 1.0770x over previous
import functools

import jax
import jax.numpy as jnp
from jax import lax
from jax.experimental import pallas as pl
from jax.experimental.pallas import tpu as pltpu
from jax.experimental.pallas import tpu_sc as plsc

N_NODES = 10000
N_EDGES = 320000
D = 128
R = 8

BN = 1000
EB = 250

CHUNK = 128
NUM_CHUNKS = N_EDGES // CHUNK
NUM_WORKERS = 32
BASE_CHUNKS = NUM_CHUNKS // NUM_WORKERS
EXTRA_CHUNKS = NUM_CHUNKS - BASE_CHUNKS * NUM_WORKERS
N_PAD = 10112
ROWS_PER_TILE = N_PAD // 16


def _tc_prep_body(x_ref, w_ref, bias_ref, h_ref, self_ref):
    xb = x_ref[...]
    for r in range(R):
        h_ref[r] = jnp.dot(xb, w_ref[r], preferred_element_type=jnp.float32)
    self_ref[...] = (
        jnp.dot(xb, w_ref[R], preferred_element_type=jnp.float32) + bias_ref[...]
    )


_tc_prep = pl.pallas_call(
    _tc_prep_body,
    grid=(N_NODES // BN,),
    in_specs=[
        pl.BlockSpec((BN, D), lambda i: (i, 0)),
        pl.BlockSpec((R + 1, D, D), lambda i: (0, 0, 0)),
        pl.BlockSpec((1, D), lambda i: (0, 0)),
    ],
    out_specs=[
        pl.BlockSpec((R, BN, D), lambda i: (0, i, 0)),
        pl.BlockSpec((BN, D), lambda i: (i, 0)),
    ],
    out_shape=[
        jax.ShapeDtypeStruct((R, N_NODES, D), jnp.float32),
        jax.ShapeDtypeStruct((N_NODES, D), jnp.float32),
    ],
)


def _tc_gidx_body(t_ref, s_ref, g_ref):
    g_ref[...] = t_ref[...] * N_NODES + s_ref[...]


_tc_gidx = pl.pallas_call(
    _tc_gidx_body,
    out_shape=jax.ShapeDtypeStruct((N_EDGES // 128, 128), jnp.int32),
)


def _tc_combine_body(acc_ref, self_ref, o_ref):
    o_ref[...] = jnp.maximum(self_ref[...] + acc_ref[0] + acc_ref[1], 0.0)


_tc_combine = pl.pallas_call(
    _tc_combine_body,
    grid=(N_NODES // BN,),
    in_specs=[
        pl.BlockSpec((2, BN, D), lambda i: (0, i, 0)),
        pl.BlockSpec((BN, D), lambda i: (i, 0)),
    ],
    out_specs=pl.BlockSpec((BN, D), lambda i: (i, 0)),
    out_shape=jax.ShapeDtypeStruct((N_NODES, D), jnp.float32),
)


@functools.cache
def _make_sc_scatter():
    @functools.partial(
        pl.kernel,
        mesh=plsc.VectorSubcoreMesh(core_axis_name="c", subcore_axis_name="s"),
        out_type=jax.ShapeDtypeStruct((2, N_PAD, D), jnp.float32),
        scratch_types=[
            pltpu.VMEM((CHUNK,), jnp.int32),
            pltpu.VMEM((CHUNK,), jnp.int32),
            pltpu.VMEM((CHUNK,), jnp.int32),
            pltpu.VMEM((CHUNK,), jnp.int32),
            pltpu.VMEM((CHUNK,), jnp.int32),
            pltpu.VMEM((CHUNK,), jnp.int32),
            pltpu.VMEM((CHUNK, D), jnp.float32),
            pltpu.VMEM((CHUNK, D), jnp.float32),
            pltpu.VMEM((CHUNK, D), jnp.float32),
            pltpu.VMEM_SHARED((N_PAD, D), jnp.float32),
            pltpu.SemaphoreType.DMA,
            pltpu.SemaphoreType.DMA,
            pltpu.SemaphoreType.DMA,
            pltpu.SemaphoreType.DMA,
            pltpu.SemaphoreType.DMA,
            pltpu.SemaphoreType.DMA,
            pltpu.SemaphoreType.DMA,
            pltpu.SemaphoreType.DMA,
            pltpu.SemaphoreType.DMA,
            pltpu.SemaphoreType.DMA,
            pltpu.SemaphoreType.DMA,
            pltpu.SemaphoreType.DMA,
        ],
    )
    def _sc_scatter(h_hbm, g_hbm, dst_hbm, zeros_hbm, acc_hbm,
                    g0, g1, g2, d0, d1, d2, r0, r1, r2, acc_sh,
                    gs0, gs1, gs2, ss0, ss1, ss2,
                    dg0, dg1, dg2, ds0, ds1, ds2):
        G = (g0, g1, g2)
        Dx = (d0, d1, d2)
        Rw = (r0, r1, r2)
        GS = (gs0, gs1, gs2)
        SS = (ss0, ss1, ss2)
        DG = (dg0, dg1, dg2)
        DS = (ds0, ds1, ds2)

        c = lax.axis_index("c")
        s = lax.axis_index("s")
        wid = s * 2 + c

        pltpu.sync_copy(
            zeros_hbm.at[pl.ds(s * ROWS_PER_TILE, ROWS_PER_TILE)],
            acc_sh.at[pl.ds(s * ROWS_PER_TILE, ROWS_PER_TILE)],
        )
        plsc.subcore_barrier()

        start = wid * BASE_CHUNKS

        def fire_load_g(j, sl):
            pltpu.async_copy(
                g_hbm.at[pl.ds((start + j) * CHUNK, CHUNK)], G[sl], DG[sl])

        def wait_load_g(j, sl):
            pltpu.make_async_copy(
                g_hbm.at[pl.ds((start + j) * CHUNK, CHUNK)], G[sl],
                DG[sl]).wait()

        def fire_load_d(j, sl):
            pltpu.async_copy(
                dst_hbm.at[pl.ds((start + j) * CHUNK, CHUNK)], Dx[sl], DS[sl])

        def wait_load_d(j, sl):
            pltpu.make_async_copy(
                dst_hbm.at[pl.ds((start + j) * CHUNK, CHUNK)], Dx[sl],
                DS[sl]).wait()

        def fire_gather(sl):
            pltpu.async_copy(h_hbm.at[G[sl]], Rw[sl], GS[sl])

        def wait_gather(sl):
            pltpu.make_async_copy(h_hbm.at[G[sl]], Rw[sl], GS[sl]).wait()

        def fire_scatter(sl):
            pltpu.async_copy(Rw[sl], acc_sh.at[Dx[sl]], SS[sl], add=True)

        def wait_scatter(sl):
            pltpu.make_async_copy(Rw[sl], acc_sh.at[Dx[sl]], SS[sl]).wait()

        fire_load_g(0, 0)
        fire_load_g(1, 1)
        fire_load_g(2, 2)
        fire_load_d(0, 0)
        fire_load_d(1, 1)
        wait_load_g(0, 0)
        fire_gather(0)
        wait_load_g(1, 1)
        fire_gather(1)
        wait_gather(0)
        wait_load_d(0, 0)
        fire_scatter(0)
        fire_load_d(2, 2)
        wait_load_g(2, 2)
        fire_gather(2)
        wait_gather(1)
        wait_load_d(1, 1)
        fire_scatter(1)
        fire_load_g(3, 0)

        def steady(j, sl):
            sp1 = (sl + 1) % 3
            sp2 = (sl + 2) % 3
            wait_scatter(sp1)
            fire_load_d(j + 1, sp1)
            wait_load_g(j + 1, sp1)
            fire_gather(sp1)
            wait_gather(sl)
            wait_load_d(j, sl)
            fire_scatter(sl)
            fire_load_g(j + 2, sp2)

        def loop_body(jj, carry):
            j = 3 * jj + 2
            steady(j, 2)
            steady(j + 1, 0)
            steady(j + 2, 1)
            return carry

        lax.fori_loop(0, (BASE_CHUNKS - 3) // 3, loop_body, 0)

        wait_scatter(0)
        wait_gather(2)
        wait_load_d(BASE_CHUNKS - 1, 2)
        fire_scatter(2)
        wait_scatter(1)
        wait_scatter(2)
        wait_load_g(BASE_CHUNKS, 0)

        @pl.when(wid < EXTRA_CHUNKS)
        def _():
            e0 = (NUM_CHUNKS - EXTRA_CHUNKS + wid) * CHUNK
            pltpu.sync_copy(g_hbm.at[pl.ds(e0, CHUNK)], G[0])
            pltpu.sync_copy(dst_hbm.at[pl.ds(e0, CHUNK)], Dx[0])
            pltpu.async_copy(h_hbm.at[G[0]], Rw[0], GS[0]).wait()
            pltpu.sync_copy(Rw[0], acc_sh.at[Dx[0]], add=True)

        plsc.subcore_barrier()
        pltpu.sync_copy(
            acc_sh.at[pl.ds(s * ROWS_PER_TILE, ROWS_PER_TILE)],
            acc_hbm.at[c].at[pl.ds(s * ROWS_PER_TILE, ROWS_PER_TILE)],
        )

    return _sc_scatter


def kernel(x, edge_index, edge_types, relation_weights, self_weight, bias):
    src = edge_index[0]
    dst = edge_index[1]
    w_all = jnp.concatenate([relation_weights, self_weight[None]], axis=0)
    bias2d = bias.reshape(1, D)
    t2d = edge_types.reshape(-1, 128)
    s2d = src.reshape(-1, 128)

    h, self_msg = _tc_prep(x, w_all, bias2d)
    g2d = _tc_gidx(t2d, s2d)
    h_flat = h.reshape(R * N_NODES, D)
    g = g2d.reshape(-1)
    zeros = jnp.zeros((N_PAD, D), jnp.float32)

    acc = _make_sc_scatter()(h_flat, g, dst, zeros)
    return _tc_combine(acc, self_msg)


# speedup vs baseline: 69.2396x; 1.0113x over previous
import functools

import jax
import jax.numpy as jnp
from jax import lax
from jax.experimental import pallas as pl
from jax.experimental.pallas import tpu as pltpu
from jax.experimental.pallas import tpu_sc as plsc

N_NODES = 10000
N_EDGES = 320000
D = 128
R = 8

BN = 2000
EB = 500
ZB = 2016

CHUNK = 128
NUM_CHUNKS = N_EDGES // CHUNK
NUM_WORKERS = 32
BASE_CHUNKS = NUM_CHUNKS // NUM_WORKERS
EXTRA_CHUNKS = NUM_CHUNKS - BASE_CHUNKS * NUM_WORKERS
N_PAD = 10112
ROWS_PER_TILE = N_PAD // 16


def _tc_prep_body(x_ref, w_ref, bias_ref, h_ref, self_ref, z_ref):
    xb = x_ref[...]
    for r in range(R):
        h_ref[r] = jnp.dot(xb, w_ref[r], preferred_element_type=jnp.float32)
    self_ref[...] = (
        jnp.dot(xb, w_ref[R], preferred_element_type=jnp.float32) + bias_ref[...]
    )
    z_ref[...] = jnp.zeros((ZB, D), jnp.float32)


_tc_prep = pl.pallas_call(
    _tc_prep_body,
    grid=(N_NODES // BN,),
    in_specs=[
        pl.BlockSpec((BN, D), lambda i: (i, 0)),
        pl.BlockSpec((R + 1, D, D), lambda i: (0, 0, 0)),
        pl.BlockSpec((1, D), lambda i: (0, 0)),
    ],
    out_specs=[
        pl.BlockSpec((R, BN, D), lambda i: (0, i, 0)),
        pl.BlockSpec((BN, D), lambda i: (i, 0)),
        pl.BlockSpec((ZB, D), lambda i: (i, 0)),
    ],
    out_shape=[
        jax.ShapeDtypeStruct((R, N_NODES, D), jnp.float32),
        jax.ShapeDtypeStruct((N_NODES, D), jnp.float32),
        jax.ShapeDtypeStruct((N_PAD, D), jnp.float32),
    ],
)


def _tc_gidx_body(t_ref, s_ref, g_ref):
    g_ref[...] = t_ref[...] * N_NODES + s_ref[...]


_tc_gidx = pl.pallas_call(
    _tc_gidx_body,
    out_shape=jax.ShapeDtypeStruct((N_EDGES // 128, 128), jnp.int32),
)


def _tc_combine_body(acc_ref, self_ref, o_ref):
    o_ref[...] = jnp.maximum(self_ref[...] + acc_ref[0] + acc_ref[1], 0.0)


_tc_combine = pl.pallas_call(
    _tc_combine_body,
    grid=(N_NODES // BN,),
    in_specs=[
        pl.BlockSpec((2, BN, D), lambda i: (0, i, 0)),
        pl.BlockSpec((BN, D), lambda i: (i, 0)),
    ],
    out_specs=pl.BlockSpec((BN, D), lambda i: (i, 0)),
    out_shape=jax.ShapeDtypeStruct((N_NODES, D), jnp.float32),
)


@functools.cache
def _make_sc_scatter():
    @functools.partial(
        pl.kernel,
        mesh=plsc.VectorSubcoreMesh(core_axis_name="c", subcore_axis_name="s"),
        out_type=jax.ShapeDtypeStruct((2, N_PAD, D), jnp.float32),
        scratch_types=[
            pltpu.VMEM((CHUNK,), jnp.int32),
            pltpu.VMEM((CHUNK,), jnp.int32),
            pltpu.VMEM((CHUNK,), jnp.int32),
            pltpu.VMEM((CHUNK,), jnp.int32),
            pltpu.VMEM((CHUNK,), jnp.int32),
            pltpu.VMEM((CHUNK,), jnp.int32),
            pltpu.VMEM((CHUNK, D), jnp.float32),
            pltpu.VMEM((CHUNK, D), jnp.float32),
            pltpu.VMEM((CHUNK, D), jnp.float32),
            pltpu.VMEM_SHARED((N_PAD, D), jnp.float32),
            pltpu.SemaphoreType.DMA,
            pltpu.SemaphoreType.DMA,
            pltpu.SemaphoreType.DMA,
            pltpu.SemaphoreType.DMA,
            pltpu.SemaphoreType.DMA,
            pltpu.SemaphoreType.DMA,
            pltpu.SemaphoreType.DMA,
            pltpu.SemaphoreType.DMA,
            pltpu.SemaphoreType.DMA,
            pltpu.SemaphoreType.DMA,
            pltpu.SemaphoreType.DMA,
            pltpu.SemaphoreType.DMA,
        ],
    )
    def _sc_scatter(h_hbm, g_hbm, dst_hbm, zeros_hbm, acc_hbm,
                    g0, g1, g2, d0, d1, d2, r0, r1, r2, acc_sh,
                    gs0, gs1, gs2, ss0, ss1, ss2,
                    dg0, dg1, dg2, ds0, ds1, ds2):
        G = (g0, g1, g2)
        Dx = (d0, d1, d2)
        Rw = (r0, r1, r2)
        GS = (gs0, gs1, gs2)
        SS = (ss0, ss1, ss2)
        DG = (dg0, dg1, dg2)
        DS = (ds0, ds1, ds2)

        c = lax.axis_index("c")
        s = lax.axis_index("s")
        wid = s * 2 + c

        pltpu.sync_copy(
            zeros_hbm.at[pl.ds(s * ROWS_PER_TILE, ROWS_PER_TILE)],
            acc_sh.at[pl.ds(s * ROWS_PER_TILE, ROWS_PER_TILE)],
        )
        plsc.subcore_barrier()

        start = wid * BASE_CHUNKS

        def fire_load_g(j, sl):
            pltpu.async_copy(
                g_hbm.at[pl.ds((start + j) * CHUNK, CHUNK)], G[sl], DG[sl])

        def wait_load_g(j, sl):
            pltpu.make_async_copy(
                g_hbm.at[pl.ds((start + j) * CHUNK, CHUNK)], G[sl],
                DG[sl]).wait()

        def fire_load_d(j, sl):
            pltpu.async_copy(
                dst_hbm.at[pl.ds((start + j) * CHUNK, CHUNK)], Dx[sl], DS[sl])

        def wait_load_d(j, sl):
            pltpu.make_async_copy(
                dst_hbm.at[pl.ds((start + j) * CHUNK, CHUNK)], Dx[sl],
                DS[sl]).wait()

        def fire_gather(sl):
            pltpu.async_copy(h_hbm.at[G[sl]], Rw[sl], GS[sl])

        def wait_gather(sl):
            pltpu.make_async_copy(h_hbm.at[G[sl]], Rw[sl], GS[sl]).wait()

        def fire_scatter(sl):
            pltpu.async_copy(Rw[sl], acc_sh.at[Dx[sl]], SS[sl], add=True)

        def wait_scatter(sl):
            pltpu.make_async_copy(Rw[sl], acc_sh.at[Dx[sl]], SS[sl]).wait()

        fire_load_g(0, 0)
        fire_load_g(1, 1)
        fire_load_g(2, 2)
        fire_load_d(0, 0)
        fire_load_d(1, 1)
        wait_load_g(0, 0)
        fire_gather(0)
        wait_load_g(1, 1)
        fire_gather(1)
        wait_gather(0)
        wait_load_d(0, 0)
        fire_scatter(0)
        fire_load_d(2, 2)
        wait_load_g(2, 2)
        fire_gather(2)
        wait_gather(1)
        wait_load_d(1, 1)
        fire_scatter(1)
        fire_load_g(3, 0)

        def steady(j, sl):
            sp1 = (sl + 1) % 3
            sp2 = (sl + 2) % 3
            wait_scatter(sp1)
            fire_load_d(j + 1, sp1)
            wait_load_g(j + 1, sp1)
            fire_gather(sp1)
            wait_gather(sl)
            wait_load_d(j, sl)
            fire_scatter(sl)
            fire_load_g(j + 2, sp2)

        def loop_body(jj, carry):
            j = 3 * jj + 2
            steady(j, 2)
            steady(j + 1, 0)
            steady(j + 2, 1)
            return carry

        lax.fori_loop(0, (BASE_CHUNKS - 3) // 3, loop_body, 0)

        wait_scatter(0)
        wait_gather(2)
        wait_load_d(BASE_CHUNKS - 1, 2)
        fire_scatter(2)
        wait_scatter(1)
        wait_scatter(2)
        wait_load_g(BASE_CHUNKS, 0)

        @pl.when(wid < EXTRA_CHUNKS)
        def _():
            e0 = (NUM_CHUNKS - EXTRA_CHUNKS + wid) * CHUNK
            pltpu.sync_copy(g_hbm.at[pl.ds(e0, CHUNK)], G[0])
            pltpu.sync_copy(dst_hbm.at[pl.ds(e0, CHUNK)], Dx[0])
            pltpu.async_copy(h_hbm.at[G[0]], Rw[0], GS[0]).wait()
            pltpu.sync_copy(Rw[0], acc_sh.at[Dx[0]], add=True)

        plsc.subcore_barrier()
        pltpu.sync_copy(
            acc_sh.at[pl.ds(s * ROWS_PER_TILE, ROWS_PER_TILE)],
            acc_hbm.at[c].at[pl.ds(s * ROWS_PER_TILE, ROWS_PER_TILE)],
        )

    return _sc_scatter


def kernel(x, edge_index, edge_types, relation_weights, self_weight, bias):
    src = edge_index[0]
    dst = edge_index[1]
    w_all = jnp.concatenate([relation_weights, self_weight[None]], axis=0)
    bias2d = bias.reshape(1, D)
    t2d = edge_types.reshape(-1, 128)
    s2d = src.reshape(-1, 128)

    h, self_msg, zeros = _tc_prep(x, w_all, bias2d)
    g2d = _tc_gidx(t2d, s2d)
    h_flat = h.reshape(R * N_NODES, D)
    g = g2d.reshape(-1)

    acc = _make_sc_scatter()(h_flat, g, dst, zeros)
    return _tc_combine(acc, self_msg)


# speedup vs baseline: 73.7881x; 1.0657x over previous
import functools

import jax
import jax.numpy as jnp
from jax import lax
from jax.experimental import pallas as pl
from jax.experimental.pallas import tpu as pltpu
from jax.experimental.pallas import tpu_sc as plsc

N_NODES = 10000
N_EDGES = 320000
D = 128
R = 8

BN = 2000
EB = 500
ZB = 2016

CHUNK = 128
NUM_CHUNKS = N_EDGES // CHUNK
NUM_WORKERS = 32
BASE_CHUNKS = NUM_CHUNKS // NUM_WORKERS
EXTRA_CHUNKS = NUM_CHUNKS - BASE_CHUNKS * NUM_WORKERS
N_PAD = 10112
ROWS_PER_TILE = N_PAD // 16


def _tc_prep_body(x_ref, w_ref, sw_ref, bias_ref, h_ref, self_ref, z_ref):
    xb = x_ref[...]
    for r in range(R):
        h_ref[r] = jnp.dot(xb, w_ref[r], preferred_element_type=jnp.float32)
    self_ref[...] = (
        jnp.dot(xb, sw_ref[...], preferred_element_type=jnp.float32)
        + bias_ref[...]
    )
    z_ref[...] = jnp.zeros((ZB, D), jnp.float32)


_tc_prep = pl.pallas_call(
    _tc_prep_body,
    grid=(N_NODES // BN,),
    in_specs=[
        pl.BlockSpec((BN, D), lambda i: (i, 0)),
        pl.BlockSpec((R, D, D), lambda i: (0, 0, 0)),
        pl.BlockSpec((D, D), lambda i: (0, 0)),
        pl.BlockSpec((1, D), lambda i: (0, 0)),
    ],
    out_specs=[
        pl.BlockSpec((R, BN, D), lambda i: (0, i, 0)),
        pl.BlockSpec((BN, D), lambda i: (i, 0)),
        pl.BlockSpec((ZB, D), lambda i: (i, 0)),
    ],
    out_shape=[
        jax.ShapeDtypeStruct((R, N_NODES, D), jnp.float32),
        jax.ShapeDtypeStruct((N_NODES, D), jnp.float32),
        jax.ShapeDtypeStruct((N_PAD, D), jnp.float32),
    ],
)


def _tc_gidx_body(t_ref, ei_ref, g_ref, d_ref):
    g_ref[...] = t_ref[...] * N_NODES + ei_ref[0]
    d_ref[...] = ei_ref[1]


_tc_gidx = pl.pallas_call(
    _tc_gidx_body,
    out_shape=[
        jax.ShapeDtypeStruct((N_EDGES // 128, 128), jnp.int32),
        jax.ShapeDtypeStruct((N_EDGES // 128, 128), jnp.int32),
    ],
)


def _tc_combine_body(acc_ref, self_ref, o_ref):
    o_ref[...] = jnp.maximum(self_ref[...] + acc_ref[0] + acc_ref[1], 0.0)


_tc_combine = pl.pallas_call(
    _tc_combine_body,
    grid=(N_NODES // BN,),
    in_specs=[
        pl.BlockSpec((2, BN, D), lambda i: (0, i, 0)),
        pl.BlockSpec((BN, D), lambda i: (i, 0)),
    ],
    out_specs=pl.BlockSpec((BN, D), lambda i: (i, 0)),
    out_shape=jax.ShapeDtypeStruct((N_NODES, D), jnp.float32),
)


@functools.cache
def _make_sc_scatter():
    @functools.partial(
        pl.kernel,
        mesh=plsc.VectorSubcoreMesh(core_axis_name="c", subcore_axis_name="s"),
        out_type=jax.ShapeDtypeStruct((2, N_PAD, D), jnp.float32),
        scratch_types=[
            pltpu.VMEM((CHUNK,), jnp.int32),
            pltpu.VMEM((CHUNK,), jnp.int32),
            pltpu.VMEM((CHUNK,), jnp.int32),
            pltpu.VMEM((CHUNK,), jnp.int32),
            pltpu.VMEM((CHUNK,), jnp.int32),
            pltpu.VMEM((CHUNK,), jnp.int32),
            pltpu.VMEM((CHUNK, D), jnp.float32),
            pltpu.VMEM((CHUNK, D), jnp.float32),
            pltpu.VMEM((CHUNK, D), jnp.float32),
            pltpu.VMEM_SHARED((N_PAD, D), jnp.float32),
            pltpu.SemaphoreType.DMA,
            pltpu.SemaphoreType.DMA,
            pltpu.SemaphoreType.DMA,
            pltpu.SemaphoreType.DMA,
            pltpu.SemaphoreType.DMA,
            pltpu.SemaphoreType.DMA,
            pltpu.SemaphoreType.DMA,
            pltpu.SemaphoreType.DMA,
            pltpu.SemaphoreType.DMA,
            pltpu.SemaphoreType.DMA,
            pltpu.SemaphoreType.DMA,
            pltpu.SemaphoreType.DMA,
        ],
    )
    def _sc_scatter(h_hbm, g_hbm, dst_hbm, zeros_hbm, acc_hbm,
                    g0, g1, g2, d0, d1, d2, r0, r1, r2, acc_sh,
                    gs0, gs1, gs2, ss0, ss1, ss2,
                    dg0, dg1, dg2, ds0, ds1, ds2):
        G = (g0, g1, g2)
        Dx = (d0, d1, d2)
        Rw = (r0, r1, r2)
        GS = (gs0, gs1, gs2)
        SS = (ss0, ss1, ss2)
        DG = (dg0, dg1, dg2)
        DS = (ds0, ds1, ds2)

        c = lax.axis_index("c")
        s = lax.axis_index("s")
        wid = s * 2 + c

        pltpu.sync_copy(
            zeros_hbm.at[pl.ds(s * ROWS_PER_TILE, ROWS_PER_TILE)],
            acc_sh.at[pl.ds(s * ROWS_PER_TILE, ROWS_PER_TILE)],
        )
        plsc.subcore_barrier()

        start = wid * BASE_CHUNKS

        def fire_load_g(j, sl):
            pltpu.async_copy(
                g_hbm.at[pl.ds((start + j) * CHUNK, CHUNK)], G[sl], DG[sl])

        def wait_load_g(j, sl):
            pltpu.make_async_copy(
                g_hbm.at[pl.ds((start + j) * CHUNK, CHUNK)], G[sl],
                DG[sl]).wait()

        def fire_load_d(j, sl):
            pltpu.async_copy(
                dst_hbm.at[pl.ds((start + j) * CHUNK, CHUNK)], Dx[sl], DS[sl])

        def wait_load_d(j, sl):
            pltpu.make_async_copy(
                dst_hbm.at[pl.ds((start + j) * CHUNK, CHUNK)], Dx[sl],
                DS[sl]).wait()

        def fire_gather(sl):
            pltpu.async_copy(h_hbm.at[G[sl]], Rw[sl], GS[sl])

        def wait_gather(sl):
            pltpu.make_async_copy(h_hbm.at[G[sl]], Rw[sl], GS[sl]).wait()

        def fire_scatter(sl):
            pltpu.async_copy(Rw[sl], acc_sh.at[Dx[sl]], SS[sl], add=True)

        def wait_scatter(sl):
            pltpu.make_async_copy(Rw[sl], acc_sh.at[Dx[sl]], SS[sl]).wait()

        fire_load_g(0, 0)
        fire_load_g(1, 1)
        fire_load_g(2, 2)
        fire_load_d(0, 0)
        fire_load_d(1, 1)
        wait_load_g(0, 0)
        fire_gather(0)
        wait_load_g(1, 1)
        fire_gather(1)
        wait_gather(0)
        wait_load_d(0, 0)
        fire_scatter(0)
        fire_load_d(2, 2)
        wait_load_g(2, 2)
        fire_gather(2)
        wait_gather(1)
        wait_load_d(1, 1)
        fire_scatter(1)
        fire_load_g(3, 0)

        def steady(j, sl):
            sp1 = (sl + 1) % 3
            sp2 = (sl + 2) % 3
            wait_scatter(sp1)
            fire_load_d(j + 1, sp1)
            wait_load_g(j + 1, sp1)
            fire_gather(sp1)
            wait_gather(sl)
            wait_load_d(j, sl)
            fire_scatter(sl)
            fire_load_g(j + 2, sp2)

        def loop_body(jj, carry):
            j = 3 * jj + 2
            steady(j, 2)
            steady(j + 1, 0)
            steady(j + 2, 1)
            return carry

        lax.fori_loop(0, (BASE_CHUNKS - 3) // 3, loop_body, 0)

        wait_scatter(0)
        wait_gather(2)
        wait_load_d(BASE_CHUNKS - 1, 2)
        fire_scatter(2)
        wait_scatter(1)
        wait_scatter(2)
        wait_load_g(BASE_CHUNKS, 0)

        @pl.when(wid < EXTRA_CHUNKS)
        def _():
            e0 = (NUM_CHUNKS - EXTRA_CHUNKS + wid) * CHUNK
            pltpu.sync_copy(g_hbm.at[pl.ds(e0, CHUNK)], G[0])
            pltpu.sync_copy(dst_hbm.at[pl.ds(e0, CHUNK)], Dx[0])
            pltpu.async_copy(h_hbm.at[G[0]], Rw[0], GS[0]).wait()
            pltpu.sync_copy(Rw[0], acc_sh.at[Dx[0]], add=True)

        plsc.subcore_barrier()
        pltpu.sync_copy(
            acc_sh.at[pl.ds(s * ROWS_PER_TILE, ROWS_PER_TILE)],
            acc_hbm.at[c].at[pl.ds(s * ROWS_PER_TILE, ROWS_PER_TILE)],
        )

    return _sc_scatter


def kernel(x, edge_index, edge_types, relation_weights, self_weight, bias):
    bias2d = bias.reshape(1, D)
    t2d = edge_types.reshape(-1, 128)
    ei3 = edge_index.reshape(2, -1, 128)

    h, self_msg, zeros = _tc_prep(x, relation_weights, self_weight, bias2d)
    g2d, d2d = _tc_gidx(t2d, ei3)
    h_flat = h.reshape(R * N_NODES, D)
    g = g2d.reshape(-1)
    dst = d2d.reshape(-1)

    acc = _make_sc_scatter()(h_flat, g, dst, zeros)
    return _tc_combine(acc, self_msg)


# speedup vs baseline: 76.4675x; 1.0363x over previous
import functools

import jax
import jax.numpy as jnp
from jax import lax
from jax.experimental import pallas as pl
from jax.experimental.pallas import tpu as pltpu
from jax.experimental.pallas import tpu_sc as plsc

N_NODES = 10000
N_EDGES = 320000
D = 128
R = 8

BN = 2000
EB = 500
ZB = 2016

CHUNK = 128
NUM_CHUNKS = N_EDGES // CHUNK
NUM_WORKERS = 32
BASE_CHUNKS = NUM_CHUNKS // NUM_WORKERS
EXTRA_CHUNKS = NUM_CHUNKS - BASE_CHUNKS * NUM_WORKERS
N_PAD = 10112
ROWS_PER_TILE = N_PAD // 16


def _tc_prep_body(x_ref, w_ref, sw_ref, bias_ref, h_ref, self_ref, z_ref):
    xb = x_ref[...]
    for r in range(R):
        h_ref[r] = jnp.dot(xb, w_ref[r], preferred_element_type=jnp.float32)
    self_ref[...] = (
        jnp.dot(xb, sw_ref[...], preferred_element_type=jnp.float32)
        + bias_ref[...]
    )
    z_ref[...] = jnp.zeros((ZB, D), jnp.float32)


_tc_prep = pl.pallas_call(
    _tc_prep_body,
    grid=(N_NODES // BN,),
    in_specs=[
        pl.BlockSpec((BN, D), lambda i: (i, 0)),
        pl.BlockSpec((R, D, D), lambda i: (0, 0, 0)),
        pl.BlockSpec((D, D), lambda i: (0, 0)),
        pl.BlockSpec((1, D), lambda i: (0, 0)),
    ],
    out_specs=[
        pl.BlockSpec((R, BN, D), lambda i: (0, i, 0)),
        pl.BlockSpec((BN, D), lambda i: (i, 0)),
        pl.BlockSpec((ZB, D), lambda i: (i, 0)),
    ],
    out_shape=[
        jax.ShapeDtypeStruct((R, N_NODES, D), jnp.float32),
        jax.ShapeDtypeStruct((N_NODES, D), jnp.float32),
        jax.ShapeDtypeStruct((N_PAD, D), jnp.float32),
    ],
)


def _tc_gidx_body(t_ref, ei_ref, g_ref, d_ref):
    g_ref[...] = t_ref[...] * N_NODES + ei_ref[0]
    d_ref[...] = ei_ref[1]


_tc_gidx = pl.pallas_call(
    _tc_gidx_body,
    out_shape=[
        jax.ShapeDtypeStruct((N_EDGES,), jnp.int32),
        jax.ShapeDtypeStruct((N_EDGES,), jnp.int32),
    ],
)


def _tc_combine_body(acc_ref, self_ref, o_ref):
    o_ref[...] = jnp.maximum(self_ref[...] + acc_ref[0] + acc_ref[1], 0.0)


_tc_combine = pl.pallas_call(
    _tc_combine_body,
    grid=(N_NODES // BN,),
    in_specs=[
        pl.BlockSpec((2, BN, D), lambda i: (0, i, 0)),
        pl.BlockSpec((BN, D), lambda i: (i, 0)),
    ],
    out_specs=pl.BlockSpec((BN, D), lambda i: (i, 0)),
    out_shape=jax.ShapeDtypeStruct((N_NODES, D), jnp.float32),
)


@functools.cache
def _make_sc_scatter():
    @functools.partial(
        pl.kernel,
        mesh=plsc.VectorSubcoreMesh(core_axis_name="c", subcore_axis_name="s"),
        out_type=jax.ShapeDtypeStruct((2, N_PAD, D), jnp.float32),
        scratch_types=[
            pltpu.VMEM((CHUNK,), jnp.int32),
            pltpu.VMEM((CHUNK,), jnp.int32),
            pltpu.VMEM((CHUNK,), jnp.int32),
            pltpu.VMEM((CHUNK,), jnp.int32),
            pltpu.VMEM((CHUNK,), jnp.int32),
            pltpu.VMEM((CHUNK,), jnp.int32),
            pltpu.VMEM((CHUNK, D), jnp.float32),
            pltpu.VMEM((CHUNK, D), jnp.float32),
            pltpu.VMEM((CHUNK, D), jnp.float32),
            pltpu.VMEM_SHARED((N_PAD, D), jnp.float32),
            pltpu.SemaphoreType.DMA,
            pltpu.SemaphoreType.DMA,
            pltpu.SemaphoreType.DMA,
            pltpu.SemaphoreType.DMA,
            pltpu.SemaphoreType.DMA,
            pltpu.SemaphoreType.DMA,
            pltpu.SemaphoreType.DMA,
            pltpu.SemaphoreType.DMA,
            pltpu.SemaphoreType.DMA,
            pltpu.SemaphoreType.DMA,
            pltpu.SemaphoreType.DMA,
            pltpu.SemaphoreType.DMA,
        ],
    )
    def _sc_scatter(h_hbm, g_hbm, dst_hbm, zeros_hbm, acc_hbm,
                    g0, g1, g2, d0, d1, d2, r0, r1, r2, acc_sh,
                    gs0, gs1, gs2, ss0, ss1, ss2,
                    dg0, dg1, dg2, ds0, ds1, ds2):
        G = (g0, g1, g2)
        Dx = (d0, d1, d2)
        Rw = (r0, r1, r2)
        GS = (gs0, gs1, gs2)
        SS = (ss0, ss1, ss2)
        DG = (dg0, dg1, dg2)
        DS = (ds0, ds1, ds2)

        c = lax.axis_index("c")
        s = lax.axis_index("s")
        wid = s * 2 + c

        pltpu.sync_copy(
            zeros_hbm.at[pl.ds(s * ROWS_PER_TILE, ROWS_PER_TILE)],
            acc_sh.at[pl.ds(s * ROWS_PER_TILE, ROWS_PER_TILE)],
        )
        plsc.subcore_barrier()

        start = wid * BASE_CHUNKS

        def fire_load_g(j, sl):
            pltpu.async_copy(
                g_hbm.at[pl.ds((start + j) * CHUNK, CHUNK)], G[sl], DG[sl])

        def wait_load_g(j, sl):
            pltpu.make_async_copy(
                g_hbm.at[pl.ds((start + j) * CHUNK, CHUNK)], G[sl],
                DG[sl]).wait()

        def fire_load_d(j, sl):
            pltpu.async_copy(
                dst_hbm.at[pl.ds((start + j) * CHUNK, CHUNK)], Dx[sl], DS[sl])

        def wait_load_d(j, sl):
            pltpu.make_async_copy(
                dst_hbm.at[pl.ds((start + j) * CHUNK, CHUNK)], Dx[sl],
                DS[sl]).wait()

        def fire_gather(sl):
            pltpu.async_copy(h_hbm.at[G[sl]], Rw[sl], GS[sl])

        def wait_gather(sl):
            pltpu.make_async_copy(h_hbm.at[G[sl]], Rw[sl], GS[sl]).wait()

        def fire_scatter(sl):
            pltpu.async_copy(Rw[sl], acc_sh.at[Dx[sl]], SS[sl], add=True)

        def wait_scatter(sl):
            pltpu.make_async_copy(Rw[sl], acc_sh.at[Dx[sl]], SS[sl]).wait()

        fire_load_g(0, 0)
        fire_load_g(1, 1)
        fire_load_g(2, 2)
        fire_load_d(0, 0)
        fire_load_d(1, 1)
        wait_load_g(0, 0)
        fire_gather(0)
        wait_load_g(1, 1)
        fire_gather(1)
        wait_gather(0)
        wait_load_d(0, 0)
        fire_scatter(0)
        fire_load_d(2, 2)
        wait_load_g(2, 2)
        fire_gather(2)
        wait_gather(1)
        wait_load_d(1, 1)
        fire_scatter(1)
        fire_load_g(3, 0)

        def steady(j, sl):
            sp1 = (sl + 1) % 3
            sp2 = (sl + 2) % 3
            wait_scatter(sp1)
            fire_load_d(j + 1, sp1)
            wait_load_g(j + 1, sp1)
            fire_gather(sp1)
            wait_gather(sl)
            wait_load_d(j, sl)
            fire_scatter(sl)
            fire_load_g(j + 2, sp2)

        def loop_body(jj, carry):
            j = 3 * jj + 2
            steady(j, 2)
            steady(j + 1, 0)
            steady(j + 2, 1)
            return carry

        lax.fori_loop(0, (BASE_CHUNKS - 3) // 3, loop_body, 0)

        wait_scatter(0)
        wait_gather(2)
        wait_load_d(BASE_CHUNKS - 1, 2)
        fire_scatter(2)
        wait_scatter(1)
        wait_scatter(2)
        wait_load_g(BASE_CHUNKS, 0)

        @pl.when(wid < EXTRA_CHUNKS)
        def _():
            e0 = (NUM_CHUNKS - EXTRA_CHUNKS + wid) * CHUNK
            pltpu.sync_copy(g_hbm.at[pl.ds(e0, CHUNK)], G[0])
            pltpu.sync_copy(dst_hbm.at[pl.ds(e0, CHUNK)], Dx[0])
            pltpu.async_copy(h_hbm.at[G[0]], Rw[0], GS[0]).wait()
            pltpu.sync_copy(Rw[0], acc_sh.at[Dx[0]], add=True)

        plsc.subcore_barrier()
        pltpu.sync_copy(
            acc_sh.at[pl.ds(s * ROWS_PER_TILE, ROWS_PER_TILE)],
            acc_hbm.at[c].at[pl.ds(s * ROWS_PER_TILE, ROWS_PER_TILE)],
        )

    return _sc_scatter


def kernel(x, edge_index, edge_types, relation_weights, self_weight, bias):
    bias2d = bias.reshape(1, D)

    h, self_msg, zeros = _tc_prep(x, relation_weights, self_weight, bias2d)
    g, dst = _tc_gidx(edge_types, edge_index)
    h_flat = h.reshape(R * N_NODES, D)

    acc = _make_sc_scatter()(h_flat, g, dst, zeros)
    return _tc_combine(acc, self_msg)


# speedup vs baseline: 77.0403x; 1.0075x over previous
import functools

import jax
import jax.numpy as jnp
from jax import lax
from jax.experimental import pallas as pl
from jax.experimental.pallas import tpu as pltpu
from jax.experimental.pallas import tpu_sc as plsc

N_NODES = 10000
N_EDGES = 320000
D = 128
R = 8

BN = 2000
EB = 500
ZB = 2016

CHUNK = 128
NUM_CHUNKS = N_EDGES // CHUNK
NUM_WORKERS = 32
BASE_CHUNKS = NUM_CHUNKS // NUM_WORKERS
EXTRA_CHUNKS = NUM_CHUNKS - BASE_CHUNKS * NUM_WORKERS
N_PAD = 10112
ROWS_PER_TILE = N_PAD // 16


EBK = N_EDGES // (N_NODES // BN)


def _tc_prep_body(x_ref, w_ref, sw_ref, bias_ref, t_ref, ei_ref,
                  h_ref, self_ref, z_ref, g_ref, d_ref):
    xb = x_ref[...]
    for r in range(R):
        h_ref[r] = jnp.dot(xb, w_ref[r], preferred_element_type=jnp.float32)
    self_ref[...] = (
        jnp.dot(xb, sw_ref[...], preferred_element_type=jnp.float32)
        + bias_ref[...]
    )
    z_ref[...] = jnp.zeros((ZB, D), jnp.float32)

    @pl.when(pl.program_id(0) == 0)
    def _():
        g_ref[...] = t_ref[...] * N_NODES + ei_ref[0]
        d_ref[...] = ei_ref[1]


_tc_prep = pl.pallas_call(
    _tc_prep_body,
    grid=(N_NODES // BN,),
    in_specs=[
        pl.BlockSpec((BN, D), lambda i: (i, 0)),
        pl.BlockSpec((R, D, D), lambda i: (0, 0, 0)),
        pl.BlockSpec((D, D), lambda i: (0, 0)),
        pl.BlockSpec((1, D), lambda i: (0, 0)),
        pl.BlockSpec((N_EDGES,), lambda i: (0,)),
        pl.BlockSpec((2, N_EDGES), lambda i: (0, 0)),
    ],
    out_specs=[
        pl.BlockSpec((R, BN, D), lambda i: (0, i, 0)),
        pl.BlockSpec((BN, D), lambda i: (i, 0)),
        pl.BlockSpec((ZB, D), lambda i: (i, 0)),
        pl.BlockSpec((N_EDGES,), lambda i: (0,)),
        pl.BlockSpec((N_EDGES,), lambda i: (0,)),
    ],
    out_shape=[
        jax.ShapeDtypeStruct((R, N_NODES, D), jnp.float32),
        jax.ShapeDtypeStruct((N_NODES, D), jnp.float32),
        jax.ShapeDtypeStruct((N_PAD, D), jnp.float32),
        jax.ShapeDtypeStruct((N_EDGES,), jnp.int32),
        jax.ShapeDtypeStruct((N_EDGES,), jnp.int32),
    ],
)


def _tc_combine_body(acc_ref, self_ref, o_ref):
    o_ref[...] = jnp.maximum(self_ref[...] + acc_ref[0] + acc_ref[1], 0.0)


_tc_combine = pl.pallas_call(
    _tc_combine_body,
    grid=(N_NODES // BN,),
    in_specs=[
        pl.BlockSpec((2, BN, D), lambda i: (0, i, 0)),
        pl.BlockSpec((BN, D), lambda i: (i, 0)),
    ],
    out_specs=pl.BlockSpec((BN, D), lambda i: (i, 0)),
    out_shape=jax.ShapeDtypeStruct((N_NODES, D), jnp.float32),
)


@functools.cache
def _make_sc_scatter():
    @functools.partial(
        pl.kernel,
        mesh=plsc.VectorSubcoreMesh(core_axis_name="c", subcore_axis_name="s"),
        out_type=jax.ShapeDtypeStruct((2, N_PAD, D), jnp.float32),
        scratch_types=[
            pltpu.VMEM((CHUNK,), jnp.int32),
            pltpu.VMEM((CHUNK,), jnp.int32),
            pltpu.VMEM((CHUNK,), jnp.int32),
            pltpu.VMEM((CHUNK,), jnp.int32),
            pltpu.VMEM((CHUNK,), jnp.int32),
            pltpu.VMEM((CHUNK,), jnp.int32),
            pltpu.VMEM((CHUNK, D), jnp.float32),
            pltpu.VMEM((CHUNK, D), jnp.float32),
            pltpu.VMEM((CHUNK, D), jnp.float32),
            pltpu.VMEM_SHARED((N_PAD, D), jnp.float32),
            pltpu.SemaphoreType.DMA,
            pltpu.SemaphoreType.DMA,
            pltpu.SemaphoreType.DMA,
            pltpu.SemaphoreType.DMA,
            pltpu.SemaphoreType.DMA,
            pltpu.SemaphoreType.DMA,
            pltpu.SemaphoreType.DMA,
            pltpu.SemaphoreType.DMA,
            pltpu.SemaphoreType.DMA,
            pltpu.SemaphoreType.DMA,
            pltpu.SemaphoreType.DMA,
            pltpu.SemaphoreType.DMA,
        ],
    )
    def _sc_scatter(h_hbm, g_hbm, dst_hbm, zeros_hbm, acc_hbm,
                    g0, g1, g2, d0, d1, d2, r0, r1, r2, acc_sh,
                    gs0, gs1, gs2, ss0, ss1, ss2,
                    dg0, dg1, dg2, ds0, ds1, ds2):
        G = (g0, g1, g2)
        Dx = (d0, d1, d2)
        Rw = (r0, r1, r2)
        GS = (gs0, gs1, gs2)
        SS = (ss0, ss1, ss2)
        DG = (dg0, dg1, dg2)
        DS = (ds0, ds1, ds2)

        c = lax.axis_index("c")
        s = lax.axis_index("s")
        wid = s * 2 + c

        pltpu.sync_copy(
            zeros_hbm.at[pl.ds(s * ROWS_PER_TILE, ROWS_PER_TILE)],
            acc_sh.at[pl.ds(s * ROWS_PER_TILE, ROWS_PER_TILE)],
        )
        plsc.subcore_barrier()

        start = wid * BASE_CHUNKS

        def fire_load_g(j, sl):
            pltpu.async_copy(
                g_hbm.at[pl.ds((start + j) * CHUNK, CHUNK)], G[sl], DG[sl])

        def wait_load_g(j, sl):
            pltpu.make_async_copy(
                g_hbm.at[pl.ds((start + j) * CHUNK, CHUNK)], G[sl],
                DG[sl]).wait()

        def fire_load_d(j, sl):
            pltpu.async_copy(
                dst_hbm.at[pl.ds((start + j) * CHUNK, CHUNK)], Dx[sl], DS[sl])

        def wait_load_d(j, sl):
            pltpu.make_async_copy(
                dst_hbm.at[pl.ds((start + j) * CHUNK, CHUNK)], Dx[sl],
                DS[sl]).wait()

        def fire_gather(sl):
            pltpu.async_copy(h_hbm.at[G[sl]], Rw[sl], GS[sl])

        def wait_gather(sl):
            pltpu.make_async_copy(h_hbm.at[G[sl]], Rw[sl], GS[sl]).wait()

        def fire_scatter(sl):
            pltpu.async_copy(Rw[sl], acc_sh.at[Dx[sl]], SS[sl], add=True)

        def wait_scatter(sl):
            pltpu.make_async_copy(Rw[sl], acc_sh.at[Dx[sl]], SS[sl]).wait()

        fire_load_g(0, 0)
        fire_load_g(1, 1)
        fire_load_g(2, 2)
        fire_load_d(0, 0)
        fire_load_d(1, 1)
        wait_load_g(0, 0)
        fire_gather(0)
        wait_load_g(1, 1)
        fire_gather(1)
        wait_gather(0)
        wait_load_d(0, 0)
        fire_scatter(0)
        fire_load_d(2, 2)
        wait_load_g(2, 2)
        fire_gather(2)
        wait_gather(1)
        wait_load_d(1, 1)
        fire_scatter(1)
        fire_load_g(3, 0)

        def steady(j, sl):
            sp1 = (sl + 1) % 3
            sp2 = (sl + 2) % 3
            wait_scatter(sp1)
            fire_load_d(j + 1, sp1)
            wait_load_g(j + 1, sp1)
            fire_gather(sp1)
            wait_gather(sl)
            wait_load_d(j, sl)
            fire_scatter(sl)
            fire_load_g(j + 2, sp2)

        def loop_body(jj, carry):
            j = 3 * jj + 2
            steady(j, 2)
            steady(j + 1, 0)
            steady(j + 2, 1)
            return carry

        lax.fori_loop(0, (BASE_CHUNKS - 3) // 3, loop_body, 0)

        wait_scatter(0)
        wait_gather(2)
        wait_load_d(BASE_CHUNKS - 1, 2)
        fire_scatter(2)
        wait_scatter(1)
        wait_scatter(2)
        wait_load_g(BASE_CHUNKS, 0)

        @pl.when(wid < EXTRA_CHUNKS)
        def _():
            e0 = (NUM_CHUNKS - EXTRA_CHUNKS + wid) * CHUNK
            pltpu.sync_copy(g_hbm.at[pl.ds(e0, CHUNK)], G[0])
            pltpu.sync_copy(dst_hbm.at[pl.ds(e0, CHUNK)], Dx[0])
            pltpu.async_copy(h_hbm.at[G[0]], Rw[0], GS[0]).wait()
            pltpu.sync_copy(Rw[0], acc_sh.at[Dx[0]], add=True)

        plsc.subcore_barrier()
        pltpu.sync_copy(
            acc_sh.at[pl.ds(s * ROWS_PER_TILE, ROWS_PER_TILE)],
            acc_hbm.at[c].at[pl.ds(s * ROWS_PER_TILE, ROWS_PER_TILE)],
        )

    return _sc_scatter


def kernel(x, edge_index, edge_types, relation_weights, self_weight, bias):
    bias2d = bias.reshape(1, D)

    h, self_msg, zeros, g, dst = _tc_prep(
        x, relation_weights, self_weight, bias2d, edge_types, edge_index)
    h_flat = h.reshape(R * N_NODES, D)

    acc = _make_sc_scatter()(h_flat, g, dst, zeros)
    return _tc_combine(acc, self_msg)
